# Initial kernel scaffold; baseline (speedup 1.0000x reference)
import jax
import jax.numpy as jnp
from jax.experimental import pallas as pl


def kernel(x, ffn_up_exps, ffn_gate_exps, ffn_down_exps, ffn_gate_inp):
    raise NotImplementedError("write your pallas kernel here")


# speedup vs baseline: 1.1272x; 1.1272x over previous
import jax
import jax.numpy as jnp
from jax import lax
from jax.experimental import pallas as pl
from jax.experimental.pallas import tpu as pltpu

T = 2048
D = 1024
H = 4096
E = 8
BLK = 128
TILES = 40
PAD = TILES * BLK
HB = 1024
NH = H // HB


def _router_kernel(x_ref, gw_ref, dest_ref, w_ref, eot_ref):
    x = x_ref[...]
    gw = gw_ref[...]
    logits = lax.dot_general(x, gw, (((1,), (1,)), ((), ())),
                             preferred_element_type=jnp.float32)
    ii = lax.broadcasted_iota(jnp.int32, (T, E), 1)
    m1 = jnp.max(logits, axis=1, keepdims=True)
    i1 = jnp.min(jnp.where(logits == m1, ii, E), axis=1, keepdims=True)
    masked = jnp.where(ii == i1, jnp.float32(-1e30), logits)
    m2 = jnp.max(masked, axis=1, keepdims=True)
    i2 = jnp.min(jnp.where(masked == m2, ii, E), axis=1, keepdims=True)
    w1 = 1.0 / (1.0 + jnp.exp(m2 - m1))
    w2 = 1.0 - w1
    oh1 = (ii == i1).astype(jnp.float32)
    oh2 = (ii == i2).astype(jnp.float32)
    oh = oh1 + oh2
    rr = lax.broadcasted_iota(jnp.int32, (T, T), 0)
    cc = lax.broadcasted_iota(jnp.int32, (T, T), 1)
    tri = (rr > cc).astype(jnp.float32)
    csum = lax.dot_general(tri, oh, (((1,), (0,)), ((), ())),
                           preferred_element_type=jnp.float32)
    counts = jnp.sum(oh, axis=0, keepdims=True)
    tiles_e = jnp.floor((counts + (BLK - 1)) * (1.0 / BLK))
    ce = lax.broadcasted_iota(jnp.int32, (E, E), 0)
    de = lax.broadcasted_iota(jnp.int32, (E, E), 1)
    mlt = (ce < de).astype(jnp.float32)
    tile_base = lax.dot_general(tiles_e, mlt, (((1,), (0,)), ((), ())),
                                preferred_element_type=jnp.float32)
    base = tile_base * float(BLK)
    pos = base + csum
    dest1 = jnp.sum(oh1 * pos, axis=1)
    dest2 = jnp.sum(oh2 * pos, axis=1)
    dest_ref[0, :] = dest1.astype(jnp.int32)
    dest_ref[1, :] = dest2.astype(jnp.int32)
    w_ref[0, :] = w1[:, 0]
    w_ref[1, :] = w2[:, 0]
    jj = lax.broadcasted_iota(jnp.int32, (64, E), 0)
    cnt = jnp.sum((tile_base.astype(jnp.int32) <= jj).astype(jnp.int32), axis=1)
    eot_ref[0, :] = cnt - 1


def _gather_kernel(dest_ref, x_ref, xs_ref):
    tt = pl.program_id(0)
    base = tt * BLK
    d1 = dest_ref[0, :]
    d2 = dest_ref[1, :]
    slot = lax.broadcasted_iota(jnp.int32, (BLK, T), 0) + base
    p = ((slot == d1[None, :]).astype(jnp.float32)
         + (slot == d2[None, :]).astype(jnp.float32))
    xs_ref[...] = lax.dot_general(p, x_ref[...], (((1,), (0,)), ((), ())),
                                  preferred_element_type=jnp.float32)


def _ffn_kernel(eot_ref, xs_ref, up_ref, gate_ref, down_ref, yout_ref, yacc_ref):
    h = pl.program_id(0)
    i = pl.program_id(1)
    xt = xs_ref[...]
    u = lax.dot_general(xt, up_ref[...], (((1,), (1,)), ((), ())),
                        preferred_element_type=jnp.float32)
    g = lax.dot_general(xt, gate_ref[...], (((1,), (1,)), ((), ())),
                        preferred_element_type=jnp.float32)
    a = g * lax.logistic(g) * u
    part = lax.dot_general(a, down_ref[...], (((1,), (1,)), ((), ())),
                           preferred_element_type=jnp.float32)
    sl = pl.ds(i * BLK, BLK)

    @pl.when(h == 0)
    def _():
        yacc_ref[sl, :] = part

    @pl.when(h != 0)
    def _():
        yacc_ref[sl, :] += part

    @pl.when(h == NH - 1)
    def _():
        yout_ref[...] = yacc_ref[sl, :]


def _scatter_kernel(dest_ref, w_ref, ys_ref, out_ref):
    tt = pl.program_id(0)
    base = tt * BLK

    @pl.when(tt == 0)
    def _():
        out_ref[...] = jnp.zeros_like(out_ref)

    d1 = dest_ref[0, :]
    d2 = dest_ref[1, :]
    w1 = w_ref[0, :]
    w2 = w_ref[1, :]
    slot = lax.broadcasted_iota(jnp.int32, (BLK, T), 0) + base
    pw = ((slot == d1[None, :]).astype(jnp.float32) * w1[None, :]
          + (slot == d2[None, :]).astype(jnp.float32) * w2[None, :])
    out_ref[...] += lax.dot_general(pw, ys_ref[...], (((0,), (0,)), ((), ())),
                                    preferred_element_type=jnp.float32)


def kernel(x, ffn_up_exps, ffn_gate_exps, ffn_down_exps, ffn_gate_inp):
    b, t, d = x.shape
    xf = x.reshape(t, d)
    dest, wts, eot2d = pl.pallas_call(
        _router_kernel,
        out_shape=[
            jax.ShapeDtypeStruct((2, T), jnp.int32),
            jax.ShapeDtypeStruct((2, T), jnp.float32),
            jax.ShapeDtypeStruct((1, 64), jnp.int32),
        ],
    )(xf, ffn_gate_inp)
    eot = eot2d[0, :TILES]

    xs = pl.pallas_call(
        _gather_kernel,
        grid=(TILES,),
        in_specs=[
            pl.BlockSpec((2, T), lambda i: (0, 0)),
            pl.BlockSpec((T, D), lambda i: (0, 0)),
        ],
        out_specs=pl.BlockSpec((BLK, D), lambda i: (i, 0)),
        out_shape=jax.ShapeDtypeStruct((PAD, D), jnp.float32),
    )(dest, xf)

    ys = pl.pallas_call(
        _ffn_kernel,
        grid_spec=pltpu.PrefetchScalarGridSpec(
            num_scalar_prefetch=1,
            grid=(NH, TILES),
            in_specs=[
                pl.BlockSpec((BLK, D), lambda h, i, eot: (i, 0)),
                pl.BlockSpec((None, HB, D), lambda h, i, eot: (eot[i], h, 0)),
                pl.BlockSpec((None, HB, D), lambda h, i, eot: (eot[i], h, 0)),
                pl.BlockSpec((None, D, HB), lambda h, i, eot: (eot[i], 0, h)),
            ],
            out_specs=pl.BlockSpec((BLK, D), lambda h, i, eot: (i, 0)),
            scratch_shapes=[pltpu.VMEM((PAD, D), jnp.float32)],
        ),
        out_shape=jax.ShapeDtypeStruct((PAD, D), jnp.float32),
    )(eot, xs, ffn_up_exps, ffn_gate_exps, ffn_down_exps)

    out = pl.pallas_call(
        _scatter_kernel,
        grid=(TILES,),
        in_specs=[
            pl.BlockSpec((2, T), lambda i: (0, 0)),
            pl.BlockSpec((2, T), lambda i: (0, 0)),
            pl.BlockSpec((BLK, D), lambda i: (i, 0)),
        ],
        out_specs=pl.BlockSpec((T, D), lambda i: (0, 0)),
        out_shape=jax.ShapeDtypeStruct((T, D), jnp.float32),
    )(dest, wts, ys)
    return out.reshape(b, t, d)
